# baseline (speedup 1.0000x reference)
import jax
import jax.numpy as jnp
from jax import lax
from jax.experimental import pallas as pl
from jax.experimental.pallas import tpu as pltpu
from jax.experimental.pallas import tpu_sc as plsc

BATCH = 16384
EMBED_DIM = 32
VROWS = 1000001
L = 16
NW = 32
B_PER_W = BATCH // NW
NCHUNK = 4
CHUNK = B_PER_W // NCHUNK


def _mf_body(user_ids, item_ids, ut, it, out_hbm,
             idx_u, idx_i, cols_u, cols_i, out_v, sem):
    wid = lax.axis_index("s") * 2 + lax.axis_index("c")
    base = wid * B_PER_W

    pltpu.sync_copy(user_ids.at[pl.ds(base, B_PER_W)], idx_u.at[0])
    pltpu.sync_copy(item_ids.at[pl.ds(base, B_PER_W)], idx_i.at[0])
    for j in range(B_PER_W // L):
        s = pl.ds(j * L, L)
        u = idx_u[0, s] + 1
        v = idx_i[0, s] + 1
        idx_u[0, s] = u
        idx_i[0, s] = v
        for d in range(1, EMBED_DIM):
            u = u + VROWS
            v = v + VROWS
            idx_u[d, s] = u
            idx_i[d, s] = v

    copies = []
    for k in range(NCHUNK):
        ks = pl.ds(k * CHUNK, CHUNK)
        for d in range(EMBED_DIM):
            copies.append(pltpu.async_copy(
                ut.at[idx_u.at[d, ks]], cols_u.at[d, ks], sem))
            copies.append(pltpu.async_copy(
                it.at[idx_i.at[d, ks]], cols_i.at[d, ks], sem))
    for c in copies:
        c.wait()

    for g in range(B_PER_W // L):
        s = pl.ds(g * L, L)
        acc = cols_u[0, s] * cols_i[0, s]
        for d in range(1, EMBED_DIM):
            acc = acc + cols_u[d, s] * cols_i[d, s]
        out_v[s] = acc

    pltpu.sync_copy(out_v, out_hbm.at[pl.ds(base, B_PER_W)])


@jax.jit
def _mf(user_ids, item_ids, user_table, item_table):
    mesh = plsc.VectorSubcoreMesh(core_axis_name="c", subcore_axis_name="s")
    return pl.kernel(
        _mf_body,
        out_type=jax.ShapeDtypeStruct((BATCH,), jnp.float32),
        mesh=mesh,
        compiler_params=pltpu.CompilerParams(
            needs_layout_passes=False, use_tc_tiling_on_sc=False),
        scratch_types=[
            pltpu.VMEM((EMBED_DIM, B_PER_W), jnp.int32),
            pltpu.VMEM((EMBED_DIM, B_PER_W), jnp.int32),
            pltpu.VMEM((EMBED_DIM, B_PER_W), jnp.float32),
            pltpu.VMEM((EMBED_DIM, B_PER_W), jnp.float32),
            pltpu.VMEM((B_PER_W,), jnp.float32),
            pltpu.SemaphoreType.DMA,
        ],
    )(user_ids, item_ids,
      user_table.T.reshape(-1), item_table.T.reshape(-1))


def kernel(user_ids, item_ids, user_table, item_table):
    return _mf(user_ids, item_ids, user_table, item_table)


# speedup vs baseline: 12.6930x; 12.6930x over previous
import jax
import jax.numpy as jnp
from jax import lax
from jax.experimental import pallas as pl
from jax.experimental.pallas import tpu as pltpu
from jax.experimental.pallas import tpu_sc as plsc

BATCH = 16384
EMBED_DIM = 32
VROWS = 1000001
VPAD = 1000064
NBLK = VPAD // 128
L = 16
NW = 32
BPW = 245
WINB = 6
WCOL = WINB * 128
NWIN = 41
GRP = 64
B_PER_W = BATCH // NW
DUMP = BATCH
STG = BATCH + GRP


def _k1_body(user_ids, item_ids, ut, it, stg_u, stg_i,
             pend_c, wl_row, wl_pos, pend_p, winbuf, rowbuf, sem):
    ids_v = pend_c
    wid = lax.axis_index("s") * 2 + lax.axis_index("c")
    lo_blk = wid * BPW
    lo_col = lo_blk * 128
    hi_col = jnp.minimum((lo_blk + BPW) * 128, VPAD)

    iota = lax.iota(jnp.int32, L)
    ones = iota >= 0
    zeros16 = jnp.zeros((L,), jnp.int32)

    for tt, ids_hbm, staging in ((ut, user_ids, stg_u), (it, item_ids, stg_i)):
        pltpu.sync_copy(ids_hbm, ids_v.at[pl.ds(0, BATCH)])

        def scan_body(j, cnt):
            row = ids_v[pl.ds(j * L, L)] + 1
            col_ok = (row >= lo_col) & (row < hi_col)
            mi = col_ok.astype(jnp.int32)
            dest = cnt + plsc.cumsum(mi) - mi
            plsc.store_scatter(wl_row, [dest], row, mask=col_ok)
            plsc.store_scatter(wl_pos, [dest], j * L + iota, mask=col_ok)
            return cnt + plsc.all_reduce_population_count(col_ok)

        cnt_v = lax.fori_loop(0, BATCH // L, scan_body, zeros16)
        plsc.store_scatter(wl_row, [cnt_v + iota], zeros16, mask=ones)
        plsc.store_scatter(wl_pos, [cnt_v + iota], DUMP + iota, mask=ones)
        ntrip = (cnt_v[0] + L - 1) // L

        def win_body(w, carry):
            ws_col = jnp.minimum(lo_col + w * WCOL, VPAD - WCOL)
            ws_col = pl.multiple_of(ws_col, 128)
            wcps = [
                pltpu.async_copy(
                    tt.at[:, pl.ds(pl.multiple_of(ws_col + b * 128, 128), 128)],
                    winbuf.at[b], sem)
                for b in range(WINB)
            ]
            for wc in wcps:
                wc.wait()

            def pend_body(v, pcnt):
                wlr = wl_row[pl.ds(v * L, L)]
                wlp = wl_pos[pl.ds(v * L, L)]
                m = (wlr >= ws_col) & (wlr < ws_col + WCOL)
                mi = m.astype(jnp.int32)
                dest = pcnt + plsc.cumsum(mi) - mi
                plsc.store_scatter(pend_c, [dest], wlr - ws_col, mask=m)
                plsc.store_scatter(pend_p, [dest >> 6, dest & (GRP - 1)],
                                   wlp, mask=m)
                return pcnt + plsc.all_reduce_population_count(m)

            pcnt_v = lax.fori_loop(0, ntrip, pend_body, zeros16)
            for sub in range(GRP // L):
                dest = pcnt_v + sub * L + iota
                plsc.store_scatter(pend_c, [dest], zeros16, mask=ones)
                plsc.store_scatter(pend_p, [dest >> 6, dest & (GRP - 1)],
                                   DUMP + sub * L + iota, mask=ones)
            gtrip = (pcnt_v[0] + GRP - 1) // GRP

            def grp_body(g, carry2):
                for sub in range(GRP // L):
                    lc = pend_c[pl.ds(g * GRP + sub * L, L)]
                    bvec = lc >> 7
                    cvec = lc & 127
                    for d in range(EMBED_DIM):
                        dvec = jnp.full((L,), d, jnp.int32)
                        val = plsc.load_gather(winbuf, [bvec, dvec, cvec])
                        plsc.store_scatter(rowbuf, [sub * L + iota, dvec],
                                           val, mask=ones)
                pltpu.async_copy(
                    rowbuf, staging.at[pend_p.at[g]], sem).wait()
                return carry2

            lax.fori_loop(0, gtrip, grp_body, 0)
            return carry

        lax.fori_loop(0, NWIN, win_body, 0)


def _k2_body(stg_u, stg_i, out_hbm, buf_u, buf_i, out_v, sem):
    wid = lax.axis_index("s") * 2 + lax.axis_index("c")
    base = wid * B_PER_W
    iota = lax.iota(jnp.int32, L)
    half = B_PER_W // 2

    for c in range(2):
        cbase = base + c * half
        cu = pltpu.async_copy(stg_u.at[pl.ds(cbase, half)], buf_u, sem)
        ci = pltpu.async_copy(stg_i.at[pl.ds(cbase, half)], buf_i, sem)
        cu.wait()
        ci.wait()

        def grp_body(g, carry):
            rows = g * L + iota
            acc = jnp.zeros((L,), jnp.float32)
            for d in range(EMBED_DIM):
                dvec = jnp.full((L,), d, jnp.int32)
                u = plsc.load_gather(buf_u, [rows, dvec])
                v = plsc.load_gather(buf_i, [rows, dvec])
                acc = acc + u * v
            out_v[pl.ds(c * half + g * L, L)] = acc
            return carry

        lax.fori_loop(0, half // L, grp_body, 0)

    pltpu.sync_copy(out_v, out_hbm.at[pl.ds(base, B_PER_W)])


@jax.jit
def _mf(user_ids, item_ids, user_table, item_table):
    mesh = plsc.VectorSubcoreMesh(core_axis_name="c", subcore_axis_name="s")
    params = pltpu.CompilerParams(
        needs_layout_passes=False, use_tc_tiling_on_sc=True)

    stg_u, stg_i = pl.kernel(
        _k1_body,
        out_type=(jax.ShapeDtypeStruct((STG, 128), jnp.float32),
                  jax.ShapeDtypeStruct((STG, 128), jnp.float32)),
        mesh=mesh,
        compiler_params=params,
        scratch_types=[
            pltpu.VMEM((BATCH + GRP,), jnp.int32),
            pltpu.VMEM((BATCH + L,), jnp.int32),
            pltpu.VMEM((BATCH + L,), jnp.int32),
            pltpu.VMEM(((BATCH + GRP) // GRP, GRP), jnp.int32),
            pltpu.VMEM((WINB, EMBED_DIM, 128), jnp.float32),
            pltpu.VMEM((GRP, 128), jnp.float32),
            pltpu.SemaphoreType.DMA,
        ],
    )(user_ids, item_ids, user_table.T, item_table.T)

    return pl.kernel(
        _k2_body,
        out_type=jax.ShapeDtypeStruct((BATCH,), jnp.float32),
        mesh=mesh,
        compiler_params=params,
        scratch_types=[
            pltpu.VMEM((B_PER_W // 2, 128), jnp.float32),
            pltpu.VMEM((B_PER_W // 2, 128), jnp.float32),
            pltpu.VMEM((B_PER_W,), jnp.float32),
            pltpu.SemaphoreType.DMA,
        ],
    )(stg_u, stg_i)


def kernel(user_ids, item_ids, user_table, item_table):
    return _mf(user_ids, item_ids, user_table, item_table)


# speedup vs baseline: 14.4566x; 1.1389x over previous
import jax
import jax.numpy as jnp
from jax import lax
from jax.experimental import pallas as pl
from jax.experimental.pallas import tpu as pltpu
from jax.experimental.pallas import tpu_sc as plsc

BATCH = 16384
EMBED_DIM = 32
VROWS = 1000001
VPAD = 1000064
NBLK = VPAD // 128
L = 16
NW = 32
BPW = 245
WINB = 8
WCOL = WINB * 128
NWIN = 31
GRP = 64
B_PER_W = BATCH // NW
DUMP = BATCH
STG = BATCH + GRP


def _k1_body(user_ids, item_ids, ut, it, stg_u, stg_i,
             pend_c, wl_row, wl_pos, pend_p, winbuf, rowbuf, sem):
    ids_v = pend_c
    wid = lax.axis_index("s") * 2 + lax.axis_index("c")
    lo_blk = wid * BPW
    lo_col = lo_blk * 128
    hi_col = jnp.minimum((lo_blk + BPW) * 128, VPAD)

    iota = lax.iota(jnp.int32, L)
    ones = iota >= 0
    zeros16 = jnp.zeros((L,), jnp.int32)

    for tt, ids_hbm, staging in ((ut, user_ids, stg_u), (it, item_ids, stg_i)):
        pltpu.sync_copy(ids_hbm, ids_v.at[pl.ds(0, BATCH)])

        def scan_body(j, cnt):
            row = ids_v[pl.ds(j * L, L)] + 1
            col_ok = (row >= lo_col) & (row < hi_col)
            mi = col_ok.astype(jnp.int32)
            dest = cnt + plsc.cumsum(mi) - mi
            plsc.store_scatter(wl_row, [dest], row, mask=col_ok)
            plsc.store_scatter(wl_pos, [dest], j * L + iota, mask=col_ok)
            return cnt + plsc.all_reduce_population_count(col_ok)

        cnt_v = lax.fori_loop(0, BATCH // L, scan_body, zeros16)
        plsc.store_scatter(wl_row, [cnt_v + iota], zeros16, mask=ones)
        plsc.store_scatter(wl_pos, [cnt_v + iota], DUMP + iota, mask=ones)
        ntrip = (cnt_v[0] + L - 1) // L

        def win_body(w, carry):
            ws_col = jnp.minimum(lo_col + w * WCOL, VPAD - WCOL)
            ws_col = pl.multiple_of(ws_col, 128)
            wcps = [
                pltpu.async_copy(
                    tt.at[:, pl.ds(pl.multiple_of(ws_col + b * 128, 128), 128)],
                    winbuf.at[b], sem)
                for b in range(WINB)
            ]
            for wc in wcps:
                wc.wait()

            def pend_body(v, pcnt):
                wlr = wl_row[pl.ds(v * L, L)]
                wlp = wl_pos[pl.ds(v * L, L)]
                m = (wlr >= ws_col) & (wlr < ws_col + WCOL)
                mi = m.astype(jnp.int32)
                dest = pcnt + plsc.cumsum(mi) - mi
                plsc.store_scatter(pend_c, [dest], wlr - ws_col, mask=m)
                plsc.store_scatter(pend_p, [dest >> 6, dest & (GRP - 1)],
                                   wlp, mask=m)
                return pcnt + plsc.all_reduce_population_count(m)

            pcnt_v = lax.fori_loop(0, ntrip, pend_body, zeros16)
            for sub in range(GRP // L):
                dest = pcnt_v + sub * L + iota
                plsc.store_scatter(pend_c, [dest], zeros16, mask=ones)
                plsc.store_scatter(pend_p, [dest >> 6, dest & (GRP - 1)],
                                   DUMP + sub * L + iota, mask=ones)
            gtrip = (pcnt_v[0] + GRP - 1) // GRP

            def grp_body(g, carry2):
                for sub in range(GRP // L):
                    lc = pend_c[pl.ds(g * GRP + sub * L, L)]
                    bvec = lc >> 7
                    cvec = lc & 127
                    for d in range(EMBED_DIM):
                        dvec = jnp.full((L,), d, jnp.int32)
                        val = plsc.load_gather(winbuf, [bvec, dvec, cvec])
                        plsc.store_scatter(rowbuf, [sub * L + iota, dvec],
                                           val, mask=ones)
                pltpu.async_copy(
                    rowbuf, staging.at[pend_p.at[g]], sem).wait()
                return carry2

            lax.fori_loop(0, gtrip, grp_body, 0)
            return carry

        lax.fori_loop(0, NWIN, win_body, 0)


def _k2_body(stg_u, stg_i, out_hbm, buf_u, buf_i, out_v, sem):
    wid = lax.axis_index("s") * 2 + lax.axis_index("c")
    base = wid * B_PER_W
    iota = lax.iota(jnp.int32, L)
    half = B_PER_W // 2

    for c in range(2):
        cbase = base + c * half
        cu = pltpu.async_copy(stg_u.at[pl.ds(cbase, half)], buf_u, sem)
        ci = pltpu.async_copy(stg_i.at[pl.ds(cbase, half)], buf_i, sem)
        cu.wait()
        ci.wait()

        def grp_body(g, carry):
            rows = g * L + iota
            acc = jnp.zeros((L,), jnp.float32)
            for d in range(EMBED_DIM):
                dvec = jnp.full((L,), d, jnp.int32)
                u = plsc.load_gather(buf_u, [rows, dvec])
                v = plsc.load_gather(buf_i, [rows, dvec])
                acc = acc + u * v
            out_v[pl.ds(c * half + g * L, L)] = acc
            return carry

        lax.fori_loop(0, half // L, grp_body, 0)

    pltpu.sync_copy(out_v, out_hbm.at[pl.ds(base, B_PER_W)])


@jax.jit
def _mf(user_ids, item_ids, user_table, item_table):
    mesh = plsc.VectorSubcoreMesh(core_axis_name="c", subcore_axis_name="s")
    params = pltpu.CompilerParams(
        needs_layout_passes=False, use_tc_tiling_on_sc=True)

    stg_u, stg_i = pl.kernel(
        _k1_body,
        out_type=(jax.ShapeDtypeStruct((STG, 128), jnp.float32),
                  jax.ShapeDtypeStruct((STG, 128), jnp.float32)),
        mesh=mesh,
        compiler_params=params,
        scratch_types=[
            pltpu.VMEM((BATCH + GRP,), jnp.int32),
            pltpu.VMEM((BATCH + L,), jnp.int32),
            pltpu.VMEM((BATCH + L,), jnp.int32),
            pltpu.VMEM(((BATCH + GRP) // GRP, GRP), jnp.int32),
            pltpu.VMEM((WINB, EMBED_DIM, 128), jnp.float32),
            pltpu.VMEM((GRP, 128), jnp.float32),
            pltpu.SemaphoreType.DMA,
        ],
    )(user_ids, item_ids, user_table.T, item_table.T)

    return pl.kernel(
        _k2_body,
        out_type=jax.ShapeDtypeStruct((BATCH,), jnp.float32),
        mesh=mesh,
        compiler_params=params,
        scratch_types=[
            pltpu.VMEM((B_PER_W // 2, 128), jnp.float32),
            pltpu.VMEM((B_PER_W // 2, 128), jnp.float32),
            pltpu.VMEM((B_PER_W,), jnp.float32),
            pltpu.SemaphoreType.DMA,
        ],
    )(stg_u, stg_i)


def kernel(user_ids, item_ids, user_table, item_table):
    return _mf(user_ids, item_ids, user_table, item_table)


# speedup vs baseline: 14.6007x; 1.0100x over previous
import jax
import jax.numpy as jnp
from jax import lax
from jax.experimental import pallas as pl
from jax.experimental.pallas import tpu as pltpu
from jax.experimental.pallas import tpu_sc as plsc

BATCH = 16384
EMBED_DIM = 32
VROWS = 1000001
VPAD = 1000064
NBLK = VPAD // 128
L = 16
NW = 32
BPW = 245
WINB = 8
WCOL = WINB * 128
NWIN = 31
GRP = 64
B_PER_W = BATCH // NW
DUMP = BATCH
STG = BATCH + GRP


def _k1_body(user_ids, item_ids, ut, it, stg_u, stg_i,
             pend_c, wl_row, wl_pos, pend_p, winbuf, rowbuf, sem):
    ids_v = pend_c
    wid = lax.axis_index("s") * 2 + lax.axis_index("c")
    lo_blk = wid * BPW
    lo_col = lo_blk * 128
    hi_col = jnp.minimum((lo_blk + BPW) * 128, VPAD)

    iota = lax.iota(jnp.int32, L)
    ones = iota >= 0
    zeros16 = jnp.zeros((L,), jnp.int32)

    for tt, ids_hbm, staging in ((ut, user_ids, stg_u), (it, item_ids, stg_i)):
        pltpu.sync_copy(ids_hbm, ids_v.at[pl.ds(0, BATCH)])

        def scan_body(j, cnt):
            row = ids_v[pl.ds(j * L, L)] + 1
            col_ok = (row >= lo_col) & (row < hi_col)
            mi = col_ok.astype(jnp.int32)
            csum = plsc.cumsum(mi)
            dest = cnt + csum - mi
            plsc.store_scatter(wl_row, [dest], row, mask=col_ok)
            plsc.store_scatter(wl_pos, [dest], j * L + iota, mask=col_ok)
            return cnt + jnp.full((L,), csum[L - 1], jnp.int32)

        cnt_v = lax.fori_loop(0, BATCH // L, scan_body, zeros16)
        plsc.store_scatter(wl_row, [cnt_v + iota], zeros16, mask=ones)
        plsc.store_scatter(wl_pos, [cnt_v + iota], DUMP + iota, mask=ones)
        ntrip = (cnt_v[0] + L - 1) // L

        def win_body(w, carry):
            ws_col = jnp.minimum(lo_col + w * WCOL, VPAD - WCOL)
            ws_col = pl.multiple_of(ws_col, 128)
            wcps = [
                pltpu.async_copy(
                    tt.at[:, pl.ds(pl.multiple_of(ws_col + b * 128, 128), 128)],
                    winbuf.at[b], sem)
                for b in range(WINB)
            ]

            def pend_body(v, pcnt):
                wlr = wl_row[pl.ds(v * L, L)]
                wlp = wl_pos[pl.ds(v * L, L)]
                m = (wlr >= ws_col) & (wlr < ws_col + WCOL)
                mi = m.astype(jnp.int32)
                csum = plsc.cumsum(mi)
                dest = pcnt + csum - mi
                plsc.store_scatter(pend_c, [dest], wlr - ws_col, mask=m)
                plsc.store_scatter(pend_p, [dest >> 6, dest & (GRP - 1)],
                                   wlp, mask=m)
                return pcnt + jnp.full((L,), csum[L - 1], jnp.int32)

            pcnt_v = lax.fori_loop(0, ntrip, pend_body, zeros16)
            for wc in wcps:
                wc.wait()
            for sub in range(GRP // L):
                dest = pcnt_v + sub * L + iota
                plsc.store_scatter(pend_c, [dest], zeros16, mask=ones)
                plsc.store_scatter(pend_p, [dest >> 6, dest & (GRP - 1)],
                                   DUMP + sub * L + iota, mask=ones)
            gtrip = (pcnt_v[0] + GRP - 1) // GRP

            def grp_body(g, carry2):
                for sub in range(GRP // L):
                    lc = pend_c[pl.ds(g * GRP + sub * L, L)]
                    bvec = lc >> 7
                    cvec = lc & 127
                    for d in range(EMBED_DIM):
                        dvec = jnp.full((L,), d, jnp.int32)
                        val = plsc.load_gather(winbuf, [bvec, dvec, cvec])
                        plsc.store_scatter(rowbuf, [sub * L + iota, dvec],
                                           val, mask=ones)
                pltpu.async_copy(
                    rowbuf, staging.at[pend_p.at[g]], sem).wait()
                return carry2

            lax.fori_loop(0, gtrip, grp_body, 0)
            return carry

        lax.fori_loop(0, NWIN, win_body, 0)


def _k2_body(stg_u, stg_i, out_hbm, buf_u, buf_i, out_v, sem):
    wid = lax.axis_index("s") * 2 + lax.axis_index("c")
    base = wid * B_PER_W
    iota = lax.iota(jnp.int32, L)
    half = B_PER_W // 2

    for c in range(2):
        cbase = base + c * half
        cu = pltpu.async_copy(stg_u.at[pl.ds(cbase, half)], buf_u, sem)
        ci = pltpu.async_copy(stg_i.at[pl.ds(cbase, half)], buf_i, sem)
        cu.wait()
        ci.wait()

        def grp_body(g, carry):
            rows = g * L + iota
            acc = jnp.zeros((L,), jnp.float32)
            for d in range(EMBED_DIM):
                dvec = jnp.full((L,), d, jnp.int32)
                u = plsc.load_gather(buf_u, [rows, dvec])
                v = plsc.load_gather(buf_i, [rows, dvec])
                acc = acc + u * v
            out_v[pl.ds(c * half + g * L, L)] = acc
            return carry

        lax.fori_loop(0, half // L, grp_body, 0)

    pltpu.sync_copy(out_v, out_hbm.at[pl.ds(base, B_PER_W)])


@jax.jit
def _mf(user_ids, item_ids, user_table, item_table):
    mesh = plsc.VectorSubcoreMesh(core_axis_name="c", subcore_axis_name="s")
    params = pltpu.CompilerParams(
        needs_layout_passes=False, use_tc_tiling_on_sc=True)

    stg_u, stg_i = pl.kernel(
        _k1_body,
        out_type=(jax.ShapeDtypeStruct((STG, 128), jnp.float32),
                  jax.ShapeDtypeStruct((STG, 128), jnp.float32)),
        mesh=mesh,
        compiler_params=params,
        scratch_types=[
            pltpu.VMEM((BATCH + GRP,), jnp.int32),
            pltpu.VMEM((BATCH + L,), jnp.int32),
            pltpu.VMEM((BATCH + L,), jnp.int32),
            pltpu.VMEM(((BATCH + GRP) // GRP, GRP), jnp.int32),
            pltpu.VMEM((WINB, EMBED_DIM, 128), jnp.float32),
            pltpu.VMEM((GRP, 128), jnp.float32),
            pltpu.SemaphoreType.DMA,
        ],
    )(user_ids, item_ids, user_table.T, item_table.T)

    return pl.kernel(
        _k2_body,
        out_type=jax.ShapeDtypeStruct((BATCH,), jnp.float32),
        mesh=mesh,
        compiler_params=params,
        scratch_types=[
            pltpu.VMEM((B_PER_W // 2, 128), jnp.float32),
            pltpu.VMEM((B_PER_W // 2, 128), jnp.float32),
            pltpu.VMEM((B_PER_W,), jnp.float32),
            pltpu.SemaphoreType.DMA,
        ],
    )(stg_u, stg_i)


def kernel(user_ids, item_ids, user_table, item_table):
    return _mf(user_ids, item_ids, user_table, item_table)


# speedup vs baseline: 15.1438x; 1.0372x over previous
import jax
import jax.numpy as jnp
from jax import lax
from jax.experimental import pallas as pl
from jax.experimental.pallas import tpu as pltpu
from jax.experimental.pallas import tpu_sc as plsc

BATCH = 16384
EMBED_DIM = 32
VROWS = 1000001
VPAD = 1000064
NBLK = VPAD // 128
L = 16
NW = 32
BPW = 245
WINB = 9
WCOL = WINB * 128
NWIN = 28
GRP = 64
B_PER_W = BATCH // NW
DUMP = BATCH
STG = BATCH + GRP


def _k1_body(user_ids, item_ids, ut, it, stg_u, stg_i,
             pend_c, wl_row, wl_pos, pend_p, winbuf, rowbuf, sem):
    ids_v = pend_c
    wid = lax.axis_index("s") * 2 + lax.axis_index("c")
    lo_blk = wid * BPW
    lo_col = lo_blk * 128
    hi_col = jnp.minimum((lo_blk + BPW) * 128, VPAD)

    iota = lax.iota(jnp.int32, L)
    ones = iota >= 0
    zeros16 = jnp.zeros((L,), jnp.int32)

    for tt, ids_hbm, staging in ((ut, user_ids, stg_u), (it, item_ids, stg_i)):
        pltpu.sync_copy(ids_hbm, ids_v.at[pl.ds(0, BATCH)])

        def scan_body(j, cnt):
            row = ids_v[pl.ds(j * L, L)] + 1
            col_ok = (row >= lo_col) & (row < hi_col)
            mi = col_ok.astype(jnp.int32)
            csum = plsc.cumsum(mi)
            dest = cnt + csum - mi
            plsc.store_scatter(wl_row, [dest], row, mask=col_ok)
            plsc.store_scatter(wl_pos, [dest], j * L + iota, mask=col_ok)
            return cnt + jnp.full((L,), csum[L - 1], jnp.int32)

        cnt_v = lax.fori_loop(0, BATCH // L, scan_body, zeros16)
        plsc.store_scatter(wl_row, [cnt_v + iota], zeros16, mask=ones)
        plsc.store_scatter(wl_pos, [cnt_v + iota], DUMP + iota, mask=ones)
        ntrip = (cnt_v[0] + L - 1) // L

        def win_body(w, carry):
            ws_col = jnp.minimum(lo_col + w * WCOL, VPAD - WCOL)
            ws_col = pl.multiple_of(ws_col, 128)
            wcps = [
                pltpu.async_copy(
                    tt.at[:, pl.ds(pl.multiple_of(ws_col + b * 128, 128), 128)],
                    winbuf.at[b], sem)
                for b in range(WINB)
            ]

            def pend_body(v, pcnt):
                wlr = wl_row[pl.ds(v * L, L)]
                wlp = wl_pos[pl.ds(v * L, L)]
                m = (wlr >= ws_col) & (wlr < ws_col + WCOL)
                mi = m.astype(jnp.int32)
                csum = plsc.cumsum(mi)
                dest = pcnt + csum - mi
                plsc.store_scatter(pend_c, [dest], wlr - ws_col, mask=m)
                plsc.store_scatter(pend_p, [dest >> 6, dest & (GRP - 1)],
                                   wlp, mask=m)
                return pcnt + jnp.full((L,), csum[L - 1], jnp.int32)

            pcnt_v = lax.fori_loop(0, ntrip, pend_body, zeros16)
            for wc in wcps:
                wc.wait()
            for sub in range(GRP // L):
                dest = pcnt_v + sub * L + iota
                plsc.store_scatter(pend_c, [dest], zeros16, mask=ones)
                plsc.store_scatter(pend_p, [dest >> 6, dest & (GRP - 1)],
                                   DUMP + sub * L + iota, mask=ones)
            gtrip = (pcnt_v[0] + GRP - 1) // GRP

            def grp_body(g, carry2):
                for sub in range(GRP // L):
                    lc = pend_c[pl.ds(g * GRP + sub * L, L)]
                    bvec = lc >> 7
                    cvec = lc & 127
                    for d in range(EMBED_DIM):
                        dvec = jnp.full((L,), d, jnp.int32)
                        val = plsc.load_gather(winbuf, [bvec, dvec, cvec])
                        plsc.store_scatter(rowbuf, [sub * L + iota, dvec],
                                           val, mask=ones)
                pltpu.async_copy(
                    rowbuf, staging.at[pend_p.at[g]], sem).wait()
                return carry2

            lax.fori_loop(0, gtrip, grp_body, 0)
            return carry

        lax.fori_loop(0, NWIN, win_body, 0)


def _k2_body(stg_u, stg_i, out_hbm, buf_u, buf_i, out_v, sem):
    wid = lax.axis_index("s") * 2 + lax.axis_index("c")
    base = wid * B_PER_W
    iota = lax.iota(jnp.int32, L)
    half = B_PER_W // 2

    for c in range(2):
        cbase = base + c * half
        cu = pltpu.async_copy(stg_u.at[pl.ds(cbase, half)], buf_u, sem)
        ci = pltpu.async_copy(stg_i.at[pl.ds(cbase, half)], buf_i, sem)
        cu.wait()
        ci.wait()

        def grp_body(g, carry):
            rows = g * L + iota
            acc = jnp.zeros((L,), jnp.float32)
            for d in range(EMBED_DIM):
                dvec = jnp.full((L,), d, jnp.int32)
                u = plsc.load_gather(buf_u, [rows, dvec])
                v = plsc.load_gather(buf_i, [rows, dvec])
                acc = acc + u * v
            out_v[pl.ds(c * half + g * L, L)] = acc
            return carry

        lax.fori_loop(0, half // L, grp_body, 0)

    pltpu.sync_copy(out_v, out_hbm.at[pl.ds(base, B_PER_W)])


@jax.jit
def _mf(user_ids, item_ids, user_table, item_table):
    mesh = plsc.VectorSubcoreMesh(core_axis_name="c", subcore_axis_name="s")
    params = pltpu.CompilerParams(
        needs_layout_passes=False, use_tc_tiling_on_sc=True)

    stg_u, stg_i = pl.kernel(
        _k1_body,
        out_type=(jax.ShapeDtypeStruct((STG, 128), jnp.float32),
                  jax.ShapeDtypeStruct((STG, 128), jnp.float32)),
        mesh=mesh,
        compiler_params=params,
        scratch_types=[
            pltpu.VMEM((BATCH + GRP,), jnp.int32),
            pltpu.VMEM((BATCH + L,), jnp.int32),
            pltpu.VMEM((BATCH + L,), jnp.int32),
            pltpu.VMEM(((BATCH + GRP) // GRP, GRP), jnp.int32),
            pltpu.VMEM((WINB, EMBED_DIM, 128), jnp.float32),
            pltpu.VMEM((GRP, 128), jnp.float32),
            pltpu.SemaphoreType.DMA,
        ],
    )(user_ids, item_ids, user_table.T, item_table.T)

    return pl.kernel(
        _k2_body,
        out_type=jax.ShapeDtypeStruct((BATCH,), jnp.float32),
        mesh=mesh,
        compiler_params=params,
        scratch_types=[
            pltpu.VMEM((B_PER_W // 2, 128), jnp.float32),
            pltpu.VMEM((B_PER_W // 2, 128), jnp.float32),
            pltpu.VMEM((B_PER_W,), jnp.float32),
            pltpu.SemaphoreType.DMA,
        ],
    )(stg_u, stg_i)


def kernel(user_ids, item_ids, user_table, item_table):
    return _mf(user_ids, item_ids, user_table, item_table)
